# baseline (speedup 1.0000x reference)
import functools

import jax
import jax.numpy as jnp
from jax import lax
from jax.experimental import pallas as pl
from jax.experimental.pallas import tpu as pltpu
from jax.experimental.pallas import tpu_sc as plsc

_DIM = 128
_MAXU = 64
_BATCH = 16384
_L = 16
_NC = 2
_NS = 16
_NW = _NC * _NS
_BPW = _BATCH // _NW
_HPW = _BATCH // _NS
_CHUNK = 128
_NCHUNK = _BPW // _CHUNK


def _body(fl_hbm, w_hbm, out_hbm,
          flh_v, hist_v, histall_v, rank_v, idx2_v, rows_v,
          w_sh, hist_sh, sem_flh, sem_flm, sem_rows, sem_g, sem_wb):
    sid = lax.axis_index("s")
    wid = sid * _NC + lax.axis_index("c")
    base = wid * _BPW

    cp_flh = pltpu.async_copy(fl_hbm.at[pl.ds(sid * _HPW, _HPW)], flh_v, sem_flh)
    cbase = lax.axis_index("c") * _BPW

    rpw = _MAXU // _NS
    pltpu.sync_copy(
        w_hbm.at[pl.ds(sid * rpw, rpw)], w_sh.at[pl.ds(sid * rpw, rpw)]
    )

    zeros = jnp.zeros((_L,), jnp.int32)
    for j in range(_MAXU // _L):
        hist_v[pl.ds(j * _L, _L)] = zeros

    ones = jnp.ones((_L,), jnp.int32)
    cp_flh.wait()
    for i in range(_HPW // _L):
        v = flh_v[pl.ds(i * _L, _L)]
        plsc.store_scatter(hist_v, [v], ones)

    pltpu.sync_copy(hist_v, hist_sh.at[sid, pl.ds(0, _MAXU)])
    plsc.subcore_barrier()
    reads = []
    for t in range(_NS):
        reads.append(
            pltpu.async_copy(
                hist_sh.at[t, pl.ds(0, _MAXU)], histall_v.at[t], sem_rows
            )
        )
    for r in reads:
        r.wait()

    running = jnp.int32(0)
    for j in range(_MAXU // _L):
        acc = zeros
        for t in range(_NS):
            acc = acc + histall_v[t, pl.ds(j * _L, _L)]
        pres = (acc > 0).astype(jnp.int32)
        inc = plsc.cumsum(pres)
        rank_v[pl.ds(j * _L, _L)] = (inc - pres) + running
        running = running + jnp.sum(pres)

    gsems = [sem_g, sem_flm]
    gathers = [None] * _NCHUNK
    wbs = []
    for g in range(_NCHUNK):
        for i in range(_CHUNK // _L):
            v = flh_v[pl.ds(cbase + g * _CHUNK + i * _L, _L)]
            r = plsc.load_gather(rank_v, [v])
            idx2_v[g, pl.ds(i * _L, _L)] = r
        if g >= 2:
            gathers[g - 2].wait()
            wbs.append(
                pltpu.async_copy(
                    rows_v.at[pl.ds((g - 2) * _CHUNK, _CHUNK)],
                    out_hbm.at[pl.ds(base + (g - 2) * _CHUNK, _CHUNK)],
                    sem_wb,
                )
            )
        gathers[g] = pltpu.async_copy(
            w_sh.at[idx2_v.at[g]],
            rows_v.at[pl.ds(g * _CHUNK, _CHUNK)],
            gsems[g % 2],
        )
    for g in range(max(_NCHUNK - 2, 0), _NCHUNK):
        gathers[g].wait()
        wbs.append(
            pltpu.async_copy(
                rows_v.at[pl.ds(g * _CHUNK, _CHUNK)],
                out_hbm.at[pl.ds(base + g * _CHUNK, _CHUNK)],
                sem_wb,
            )
        )
    for c in wbs:
        c.wait()


def kernel(file_list, W):
    mesh = plsc.VectorSubcoreMesh(
        core_axis_name="c", subcore_axis_name="s", num_cores=_NC, num_subcores=_NS
    )
    run = functools.partial(
        pl.kernel,
        out_type=jax.ShapeDtypeStruct((_BATCH, _DIM), jnp.float32),
        mesh=mesh,
        scratch_types=[
            pltpu.VMEM((_HPW,), jnp.int32),
            pltpu.VMEM((_MAXU,), jnp.int32),
            pltpu.VMEM((_NS, _MAXU), jnp.int32),
            pltpu.VMEM((_MAXU,), jnp.int32),
            pltpu.VMEM((_NCHUNK, _CHUNK), jnp.int32),
            pltpu.VMEM((_BPW, _DIM), jnp.float32),
            pltpu.MemorySpace.VMEM_SHARED((_MAXU, _DIM), jnp.float32),
            pltpu.MemorySpace.VMEM_SHARED((_NS, 256), jnp.int32),
            pltpu.SemaphoreType.DMA,
            pltpu.SemaphoreType.DMA,
            pltpu.SemaphoreType.DMA,
            pltpu.SemaphoreType.DMA,
            pltpu.SemaphoreType.DMA,
        ],
        compiler_params=pltpu.CompilerParams(needs_layout_passes=False),
    )(_body)
    return run(file_list, W)


# speedup vs baseline: 1.0104x; 1.0104x over previous
import functools

import jax
import jax.numpy as jnp
from jax import lax
from jax.experimental import pallas as pl
from jax.experimental.pallas import tpu as pltpu
from jax.experimental.pallas import tpu_sc as plsc

_DIM = 128
_MAXU = 64
_BATCH = 16384
_L = 16
_NC = 2
_NS = 16
_NW = _NC * _NS
_BPW = _BATCH // _NW
_HPW = _BATCH // _NS
_CHUNK = 128
_NCHUNK = _BPW // _CHUNK


def _body(fl_hbm, w_hbm, out_hbm,
          flh_v, hist_v, histall_v, rank_v, idx2_v, rows_v,
          w_sh, hist_sh, sem_flh, sem_flm, sem_rows, sem_g, sem_wb):
    sid = lax.axis_index("s")
    wid = sid * _NC + lax.axis_index("c")
    base = wid * _BPW

    cp_flh = pltpu.async_copy(fl_hbm.at[pl.ds(sid * _HPW, _HPW)], flh_v, sem_flh)
    cbase = lax.axis_index("c") * _BPW

    rpw = _MAXU // _NS
    pltpu.sync_copy(
        w_hbm.at[pl.ds(sid * rpw, rpw)], w_sh.at[pl.ds(sid * rpw, rpw)]
    )

    zeros = jnp.zeros((_L,), jnp.int32)
    for j in range(_MAXU // _L):
        hist_v[pl.ds(j * _L, _L)] = zeros

    ones = jnp.ones((_L,), jnp.int32)
    cp_flh.wait()
    for i in range(_HPW // _L):
        v = flh_v[pl.ds(i * _L, _L)]
        plsc.store_scatter(hist_v, [v], ones)

    pltpu.sync_copy(hist_v, hist_sh.at[sid, pl.ds(0, _MAXU)])
    plsc.subcore_barrier()
    pltpu.sync_copy(hist_sh, histall_v)

    running = jnp.int32(0)
    for j in range(_MAXU // _L):
        acc = zeros
        for t in range(_NS):
            acc = acc + histall_v[t, pl.ds(j * _L, _L)]
        pres = (acc > 0).astype(jnp.int32)
        inc = plsc.cumsum(pres)
        rank_v[pl.ds(j * _L, _L)] = (inc - pres) + running
        running = running + jnp.sum(pres)

    gsems = [sem_flh, sem_flm, sem_rows, sem_g]
    gathers = [None] * _NCHUNK
    for g in range(_NCHUNK):
        for i in range(_CHUNK // _L):
            v = flh_v[pl.ds(cbase + g * _CHUNK + i * _L, _L)]
            r = plsc.load_gather(rank_v, [v])
            idx2_v[g, pl.ds(i * _L, _L)] = r
        gathers[g] = pltpu.async_copy(
            w_sh.at[idx2_v.at[g]],
            rows_v.at[pl.ds(g * _CHUNK, _CHUNK)],
            gsems[g % len(gsems)],
        )
    wbs = []
    for g in range(_NCHUNK):
        gathers[g].wait()
        wbs.append(
            pltpu.async_copy(
                rows_v.at[pl.ds(g * _CHUNK, _CHUNK)],
                out_hbm.at[pl.ds(base + g * _CHUNK, _CHUNK)],
                sem_wb,
            )
        )
    for c in wbs:
        c.wait()


def kernel(file_list, W):
    mesh = plsc.VectorSubcoreMesh(
        core_axis_name="c", subcore_axis_name="s", num_cores=_NC, num_subcores=_NS
    )
    run = functools.partial(
        pl.kernel,
        out_type=jax.ShapeDtypeStruct((_BATCH, _DIM), jnp.float32),
        mesh=mesh,
        scratch_types=[
            pltpu.VMEM((_HPW,), jnp.int32),
            pltpu.VMEM((_MAXU,), jnp.int32),
            pltpu.VMEM((_NS, 256), jnp.int32),
            pltpu.VMEM((_MAXU,), jnp.int32),
            pltpu.VMEM((_NCHUNK, _CHUNK), jnp.int32),
            pltpu.VMEM((_BPW, _DIM), jnp.float32),
            pltpu.MemorySpace.VMEM_SHARED((_MAXU, _DIM), jnp.float32),
            pltpu.MemorySpace.VMEM_SHARED((_NS, 256), jnp.int32),
            pltpu.SemaphoreType.DMA,
            pltpu.SemaphoreType.DMA,
            pltpu.SemaphoreType.DMA,
            pltpu.SemaphoreType.DMA,
            pltpu.SemaphoreType.DMA,
        ],
        compiler_params=pltpu.CompilerParams(needs_layout_passes=False),
    )(_body)
    return run(file_list, W)


# speedup vs baseline: 1.0107x; 1.0003x over previous
import functools

import jax
import jax.numpy as jnp
from jax import lax
from jax.experimental import pallas as pl
from jax.experimental.pallas import tpu as pltpu
from jax.experimental.pallas import tpu_sc as plsc

_DIM = 128
_MAXU = 64
_BATCH = 16384
_L = 16
_NC = 2
_NS = 16
_NW = _NC * _NS
_BPW = _BATCH // _NW
_HPW = _BATCH // _NS
_CHUNK = 128
_NCHUNK = _BPW // _CHUNK


def _body(fl_hbm, w_hbm, out_hbm,
          flh_v, hist_v, histall_v, rank_v, idx2_v, rows_v,
          w_sh, hist_sh, sem_flh, sem_flm, sem_rows, sem_g, sem_wb):
    sid = lax.axis_index("s")
    wid = sid * _NC + lax.axis_index("c")
    base = wid * _BPW

    tr0 = jax.named_scope("ph_stage"); tr0.__enter__()
    cp_flh = pltpu.async_copy(fl_hbm.at[pl.ds(sid * _HPW, _HPW)], flh_v, sem_flh)
    cbase = lax.axis_index("c") * _BPW

    rpw = _MAXU // _NS
    pltpu.sync_copy(
        w_hbm.at[pl.ds(sid * rpw, rpw)], w_sh.at[pl.ds(sid * rpw, rpw)]
    )

    zeros = jnp.zeros((_L,), jnp.int32)
    for j in range(_MAXU // _L):
        hist_v[pl.ds(j * _L, _L)] = zeros

    ones = jnp.ones((_L,), jnp.int32)
    cp_flh.wait()
    tr0.__exit__(None, None, None)
    tr1 = jax.named_scope("ph_hist"); tr1.__enter__()
    for i in range(_HPW // _L):
        v = flh_v[pl.ds(i * _L, _L)]
        plsc.store_scatter(hist_v, [v], ones)

    tr1.__exit__(None, None, None)
    tr2 = jax.named_scope("ph_merge"); tr2.__enter__()
    pltpu.sync_copy(hist_v, hist_sh.at[sid, pl.ds(0, _MAXU)])
    plsc.subcore_barrier()
    pltpu.sync_copy(hist_sh, histall_v)

    running = jnp.int32(0)
    for j in range(_MAXU // _L):
        acc = zeros
        for t in range(_NS):
            acc = acc + histall_v[t, pl.ds(j * _L, _L)]
        pres = (acc > 0).astype(jnp.int32)
        inc = plsc.cumsum(pres)
        rank_v[pl.ds(j * _L, _L)] = (inc - pres) + running
        running = running + jnp.sum(pres)

    tr2.__exit__(None, None, None)
    tr3 = jax.named_scope("ph_gather"); tr3.__enter__()
    gsems = [sem_flh, sem_flm, sem_rows, sem_g]
    gathers = [None] * _NCHUNK
    for g in range(_NCHUNK):
        for i in range(_CHUNK // _L):
            v = flh_v[pl.ds(cbase + g * _CHUNK + i * _L, _L)]
            r = plsc.load_gather(rank_v, [v])
            idx2_v[g, pl.ds(i * _L, _L)] = r
        gathers[g] = pltpu.async_copy(
            w_sh.at[idx2_v.at[g]],
            rows_v.at[pl.ds(g * _CHUNK, _CHUNK)],
            gsems[g % len(gsems)],
        )
    wbs = []
    for g in range(_NCHUNK):
        gathers[g].wait()
        wbs.append(
            pltpu.async_copy(
                rows_v.at[pl.ds(g * _CHUNK, _CHUNK)],
                out_hbm.at[pl.ds(base + g * _CHUNK, _CHUNK)],
                sem_wb,
            )
        )
    tr3.__exit__(None, None, None)
    tr4 = jax.named_scope("ph_drain"); tr4.__enter__()
    for c in wbs:
        c.wait()
    tr4.__exit__(None, None, None)


def kernel(file_list, W):
    mesh = plsc.VectorSubcoreMesh(
        core_axis_name="c", subcore_axis_name="s", num_cores=_NC, num_subcores=_NS
    )
    run = functools.partial(
        pl.kernel,
        out_type=jax.ShapeDtypeStruct((_BATCH, _DIM), jnp.float32),
        mesh=mesh,
        scratch_types=[
            pltpu.VMEM((_HPW,), jnp.int32),
            pltpu.VMEM((_MAXU,), jnp.int32),
            pltpu.VMEM((_NS, 256), jnp.int32),
            pltpu.VMEM((_MAXU,), jnp.int32),
            pltpu.VMEM((_NCHUNK, _CHUNK), jnp.int32),
            pltpu.VMEM((_BPW, _DIM), jnp.float32),
            pltpu.MemorySpace.VMEM_SHARED((_MAXU, _DIM), jnp.float32),
            pltpu.MemorySpace.VMEM_SHARED((_NS, 256), jnp.int32),
            pltpu.SemaphoreType.DMA,
            pltpu.SemaphoreType.DMA,
            pltpu.SemaphoreType.DMA,
            pltpu.SemaphoreType.DMA,
            pltpu.SemaphoreType.DMA,
        ],
        compiler_params=pltpu.CompilerParams(needs_layout_passes=False),
    )(_body)
    return run(file_list, W)


# speedup vs baseline: 1.0229x; 1.0121x over previous
import functools

import jax
import jax.numpy as jnp
from jax import lax
from jax.experimental import pallas as pl
from jax.experimental.pallas import tpu as pltpu
from jax.experimental.pallas import tpu_sc as plsc

_DIM = 128
_MAXU = 64
_BATCH = 16384
_L = 16
_NC = 2
_NS = 16
_NW = _NC * _NS
_BPW = _BATCH // _NW
_HPW = _BATCH // _NS
_CHUNK = 64
_NCHUNK = _BPW // _CHUNK


def _body(fl_hbm, w_hbm, out_hbm,
          flh_v, hist_v, histall_v, rank_v, idx2_v, rows_v,
          w_sh, hist_sh, sem_flh, sem_flm, sem_rows, sem_g,
          sem_g4, sem_g5, sem_g6, sem_g7, sem_wb):
    sid = lax.axis_index("s")
    wid = sid * _NC + lax.axis_index("c")
    base = wid * _BPW

    cp_flh = pltpu.async_copy(fl_hbm.at[pl.ds(sid * _HPW, _HPW)], flh_v, sem_flh)
    cbase = lax.axis_index("c") * _BPW

    rpw = _MAXU // _NS
    pltpu.sync_copy(
        w_hbm.at[pl.ds(sid * rpw, rpw)], w_sh.at[pl.ds(sid * rpw, rpw)]
    )

    zeros = jnp.zeros((_L,), jnp.int32)
    for j in range(_MAXU // _L):
        hist_v[pl.ds(j * _L, _L)] = zeros

    ones = jnp.ones((_L,), jnp.int32)
    cp_flh.wait()
    for i in range(_HPW // _L):
        v = flh_v[pl.ds(i * _L, _L)]
        plsc.store_scatter(hist_v, [v], ones)

    pltpu.sync_copy(hist_v, hist_sh.at[sid, pl.ds(0, _MAXU)])
    plsc.subcore_barrier()
    pltpu.sync_copy(hist_sh, histall_v)

    running = jnp.int32(0)
    for j in range(_MAXU // _L):
        acc = zeros
        for t in range(_NS):
            acc = acc + histall_v[t, pl.ds(j * _L, _L)]
        pres = (acc > 0).astype(jnp.int32)
        inc = plsc.cumsum(pres)
        rank_v[pl.ds(j * _L, _L)] = (inc - pres) + running
        running = running + jnp.sum(pres)

    gsems = [sem_flh, sem_flm, sem_rows, sem_g, sem_g4, sem_g5, sem_g6, sem_g7]
    gathers = [None] * _NCHUNK
    for g in range(_NCHUNK):
        for i in range(_CHUNK // _L):
            v = flh_v[pl.ds(cbase + g * _CHUNK + i * _L, _L)]
            r = plsc.load_gather(rank_v, [v])
            idx2_v[g, pl.ds(i * _L, _L)] = r
        gathers[g] = pltpu.async_copy(
            w_sh.at[idx2_v.at[g]],
            rows_v.at[pl.ds(g * _CHUNK, _CHUNK)],
            gsems[g % len(gsems)],
        )
    wbs = []
    for g in range(_NCHUNK):
        gathers[g].wait()
        wbs.append(
            pltpu.async_copy(
                rows_v.at[pl.ds(g * _CHUNK, _CHUNK)],
                out_hbm.at[pl.ds(base + g * _CHUNK, _CHUNK)],
                sem_wb,
            )
        )
    for c in wbs:
        c.wait()


def kernel(file_list, W):
    mesh = plsc.VectorSubcoreMesh(
        core_axis_name="c", subcore_axis_name="s", num_cores=_NC, num_subcores=_NS
    )
    run = functools.partial(
        pl.kernel,
        out_type=jax.ShapeDtypeStruct((_BATCH, _DIM), jnp.float32),
        mesh=mesh,
        scratch_types=[
            pltpu.VMEM((_HPW,), jnp.int32),
            pltpu.VMEM((_MAXU,), jnp.int32),
            pltpu.VMEM((_NS, 256), jnp.int32),
            pltpu.VMEM((_MAXU,), jnp.int32),
            pltpu.VMEM((_NCHUNK, _CHUNK), jnp.int32),
            pltpu.VMEM((_BPW, _DIM), jnp.float32),
            pltpu.MemorySpace.VMEM_SHARED((_MAXU, _DIM), jnp.float32),
            pltpu.MemorySpace.VMEM_SHARED((_NS, 256), jnp.int32),
            pltpu.SemaphoreType.DMA,
            pltpu.SemaphoreType.DMA,
            pltpu.SemaphoreType.DMA,
            pltpu.SemaphoreType.DMA,
            pltpu.SemaphoreType.DMA,
            pltpu.SemaphoreType.DMA,
            pltpu.SemaphoreType.DMA,
            pltpu.SemaphoreType.DMA,
            pltpu.SemaphoreType.DMA,
        ],
        compiler_params=pltpu.CompilerParams(needs_layout_passes=False),
    )(_body)
    return run(file_list, W)
